# Initial kernel scaffold; baseline (speedup 1.0000x reference)
import jax
import jax.numpy as jnp
from jax.experimental import pallas as pl


def kernel(indices, table, W, b):
    raise NotImplementedError("write your pallas kernel here")


# speedup vs baseline: 3.2969x; 3.2969x over previous
import functools

import jax
import jax.numpy as jnp
from jax import lax
from jax.experimental import pallas as pl
from jax.experimental.pallas import tpu as pltpu
from jax.experimental.pallas import tpu_sc as plsc

_EPS = 1e-12

_NC = 2
_NS = 16
_NW = _NC * _NS
_STREAM = 128


def _sc_gather_kernel(rows_per_worker, dp, idx_hbm, table_hbm, out_hbm,
                      idx_v, rows_v, sem):
    wid = lax.axis_index("s") * _NC + lax.axis_index("c")
    n_streams = rows_per_worker // _STREAM
    row0 = wid * n_streams
    pltpu.sync_copy(idx_hbm.at[pl.ds(row0, n_streams)], idx_v)

    def body(j, carry):
        pltpu.async_copy(table_hbm.at[idx_v.at[j]], rows_v, sem).wait()
        pltpu.sync_copy(rows_v, out_hbm.at[pl.ds((row0 + j) * _STREAM,
                                                 _STREAM)])
        return carry

    lax.fori_loop(0, n_streams, body, 0, unroll=False)


def _sc_gather(idx2, table_p):
    n_rows = idx2.shape[0] * idx2.shape[1]
    dp = table_p.shape[1]
    rows_per_worker = n_rows // _NW
    n_streams = rows_per_worker // _STREAM
    mesh = plsc.VectorSubcoreMesh(core_axis_name="c", subcore_axis_name="s")
    kern = pl.kernel(
        functools.partial(_sc_gather_kernel, rows_per_worker, dp),
        out_type=jax.ShapeDtypeStruct((n_rows, dp), jnp.float32),
        mesh=mesh,
        scratch_types=[
            pltpu.VMEM((n_streams, _STREAM), jnp.int32),
            pltpu.VMEM((_STREAM, dp), jnp.float32),
            pltpu.SemaphoreType.DMA,
        ],
        compiler_params=pltpu.CompilerParams(use_tc_tiling_on_sc=False),
    )
    return kern(idx2, table_p)


def _tc_dense_kernel(emb_ref, wt_ref, b_ref, out_ref):
    x = emb_ref[...]
    n = jnp.sqrt(jnp.sum(x * x, axis=1, keepdims=True))
    x = x / jnp.maximum(n, _EPS)
    x = jnp.tanh(x)
    h = jnp.dot(x, wt_ref[...], preferred_element_type=jnp.float32)
    h = h + b_ref[...]
    hn = jnp.sqrt(jnp.sum(h * h, axis=1, keepdims=True))
    h = h / jnp.maximum(hn, _EPS)
    out_ref[...] = jnp.maximum(h, 0.0)


def _tc_dense(emb, wt, b2, block_n):
    n_rows, dp = emb.shape
    out_dim = wt.shape[1]
    grid = (n_rows // block_n,)
    return pl.pallas_call(
        _tc_dense_kernel,
        grid=grid,
        in_specs=[
            pl.BlockSpec((block_n, dp), lambda i: (i, 0)),
            pl.BlockSpec((dp, out_dim), lambda i: (0, 0)),
            pl.BlockSpec((1, out_dim), lambda i: (0, 0)),
        ],
        out_specs=pl.BlockSpec((block_n, out_dim), lambda i: (i, 0)),
        out_shape=jax.ShapeDtypeStruct((n_rows, out_dim), jnp.float32),
    )(emb, wt, b2)


@jax.jit
def kernel(indices, table, W, b):
    n = indices.shape[0]
    v, d = table.shape
    out_dim = W.shape[0]
    dp = 32
    table_p = jnp.pad(table, ((0, 0), (0, dp - d)))
    wt = jnp.pad(W, ((0, 0), (0, dp - d))).T
    b2 = b.reshape(1, out_dim)
    idx2 = indices.astype(jnp.int32).reshape(n // _STREAM, _STREAM)
    emb = _sc_gather(idx2, table_p)
    return _tc_dense(emb, wt, b2, block_n=2048)


# speedup vs baseline: 6.7270x; 2.0404x over previous
import functools

import jax
import jax.numpy as jnp
from jax import lax
from jax.experimental import pallas as pl
from jax.experimental.pallas import tpu as pltpu
from jax.experimental.pallas import tpu_sc as plsc

_EPS = 1e-12

_NC = 2
_NS = 16
_NW = _NC * _NS
_STREAM = 128


def _sc_gather_kernel(n_streams, idx_hbm, table_hbm, out_hbm,
                      idx_v, rows0, rows1, ga, gb):
    wid = lax.axis_index("s") * _NC + lax.axis_index("c")
    row0 = wid * n_streams
    pltpu.sync_copy(idx_hbm.at[pl.ds(row0, n_streams)], idx_v)

    def out_at(j):
        return out_hbm.at[pl.ds((row0 + j) * _STREAM, _STREAM)]

    pltpu.async_copy(table_hbm.at[idx_v.at[0]], rows0, ga)

    def pair(p, carry):
        j0 = 2 * p
        pltpu.async_copy(table_hbm.at[idx_v.at[j0 + 1]], rows1, gb)
        pltpu.make_async_copy(table_hbm.at[idx_v.at[j0]], rows0, ga).wait()
        pltpu.sync_copy(rows0, out_at(j0))

        @pl.when(j0 + 2 < n_streams)
        def _():
            pltpu.async_copy(table_hbm.at[idx_v.at[j0 + 2]], rows0, ga)

        pltpu.make_async_copy(table_hbm.at[idx_v.at[j0 + 1]], rows1,
                              gb).wait()
        pltpu.sync_copy(rows1, out_at(j0 + 1))
        return carry

    lax.fori_loop(0, n_streams // 2, pair, 0, unroll=False)


def _sc_gather(idx2, table):
    n_rows = idx2.shape[0] * idx2.shape[1]
    d = table.shape[1]
    n_streams = n_rows // (_NW * _STREAM)
    mesh = plsc.VectorSubcoreMesh(core_axis_name="c", subcore_axis_name="s")
    kern = pl.kernel(
        functools.partial(_sc_gather_kernel, n_streams),
        out_type=jax.ShapeDtypeStruct((n_rows, d), jnp.float32),
        mesh=mesh,
        scratch_types=[
            pltpu.VMEM((n_streams, _STREAM), jnp.int32),
            pltpu.VMEM((_STREAM, d), jnp.float32),
            pltpu.VMEM((_STREAM, d), jnp.float32),
            pltpu.SemaphoreType.DMA,
            pltpu.SemaphoreType.DMA,
        ],
        compiler_params=pltpu.CompilerParams(use_tc_tiling_on_sc=False),
    )
    return kern(idx2, table)


def _tc_dense_kernel(emb_ref, wt_ref, b_ref, out_ref):
    x = emb_ref[...]
    n = jnp.sqrt(jnp.sum(x * x, axis=1, keepdims=True))
    x = x / jnp.maximum(n, _EPS)
    x = jnp.tanh(x)
    h = jnp.dot(x, wt_ref[...], preferred_element_type=jnp.float32)
    h = h + b_ref[...]
    hn = jnp.sqrt(jnp.sum(h * h, axis=1, keepdims=True))
    h = h / jnp.maximum(hn, _EPS)
    out_ref[...] = jnp.maximum(h, 0.0)


def _tc_dense(emb, wt, b2, block_n):
    n_rows, dp = emb.shape
    out_dim = wt.shape[1]
    grid = (n_rows // block_n,)
    return pl.pallas_call(
        _tc_dense_kernel,
        grid=grid,
        in_specs=[
            pl.BlockSpec((block_n, dp), lambda i: (i, 0)),
            pl.BlockSpec((dp, out_dim), lambda i: (0, 0)),
            pl.BlockSpec((1, out_dim), lambda i: (0, 0)),
        ],
        out_specs=pl.BlockSpec((block_n, out_dim), lambda i: (i, 0)),
        out_shape=jax.ShapeDtypeStruct((n_rows, out_dim), jnp.float32),
    )(emb, wt, b2)


@jax.jit
def kernel(indices, table, W, b):
    n = indices.shape[0]
    v, d = table.shape
    out_dim = W.shape[0]
    dp = 32
    block_v = 2048
    vp = (v + block_v - 1) // block_v * block_v
    table_p = jnp.pad(table, ((0, vp - v), (0, dp - d)))
    wt = jnp.pad(W, ((0, 0), (0, dp - d))).T
    b2 = b.reshape(1, out_dim)
    t4 = _tc_dense(table_p, wt, b2, block_n=block_v)
    idx2 = indices.astype(jnp.int32).reshape(n // _STREAM, _STREAM)
    return _sc_gather(idx2, t4)


# speedup vs baseline: 7.2975x; 1.0848x over previous
import functools

import jax
import jax.numpy as jnp
from jax import lax
from jax.experimental import pallas as pl
from jax.experimental.pallas import tpu as pltpu
from jax.experimental.pallas import tpu_sc as plsc

_EPS = 1e-12

_NC = 2
_NS = 16
_NW = _NC * _NS
_STREAM = 128


def _sc_gather_kernel(n_streams, idx_hbm, table_hbm, out_hbm, idx_v,
                      r0, r1, r2, r3, g0, g1, g2, g3):
    wid = lax.axis_index("s") * _NC + lax.axis_index("c")
    row0 = wid * n_streams
    pltpu.sync_copy(idx_hbm.at[pl.ds(row0, n_streams)], idx_v)

    def fire(j, buf, sem):
        pltpu.async_copy(table_hbm.at[idx_v.at[j]], buf, sem)

    def fire_if(j, buf, sem):
        @pl.when(j < n_streams)
        def _():
            fire(j, buf, sem)

    def drain_write(j, buf, sem):
        pltpu.make_async_copy(table_hbm.at[idx_v.at[0]], buf, sem).wait()
        pltpu.sync_copy(buf, out_hbm.at[pl.ds((row0 + j) * _STREAM,
                                              _STREAM)])

    fire(0, r0, g0)
    fire(1, r1, g1)

    def quad(q, carry):
        j = 4 * q
        fire(j + 2, r2, g2)
        drain_write(j, r0, g0)
        fire(j + 3, r3, g3)
        drain_write(j + 1, r1, g1)
        fire_if(j + 4, r0, g0)
        drain_write(j + 2, r2, g2)
        fire_if(j + 5, r1, g1)
        drain_write(j + 3, r3, g3)
        return carry

    lax.fori_loop(0, n_streams // 4, quad, 0, unroll=False)


def _sc_gather(idx2, table):
    n_rows = idx2.shape[0] * idx2.shape[1]
    d = table.shape[1]
    n_streams = n_rows // (_NW * _STREAM)
    mesh = plsc.VectorSubcoreMesh(core_axis_name="c", subcore_axis_name="s")
    kern = pl.kernel(
        functools.partial(_sc_gather_kernel, n_streams),
        out_type=jax.ShapeDtypeStruct((n_rows, d), jnp.float32),
        mesh=mesh,
        scratch_types=[
            pltpu.VMEM((n_streams, _STREAM), jnp.int32),
            pltpu.VMEM((_STREAM, d), jnp.float32),
            pltpu.VMEM((_STREAM, d), jnp.float32),
            pltpu.VMEM((_STREAM, d), jnp.float32),
            pltpu.VMEM((_STREAM, d), jnp.float32),
            pltpu.SemaphoreType.DMA,
            pltpu.SemaphoreType.DMA,
            pltpu.SemaphoreType.DMA,
            pltpu.SemaphoreType.DMA,
        ],
        compiler_params=pltpu.CompilerParams(use_tc_tiling_on_sc=False),
    )
    return kern(idx2, table)


def _tc_dense_kernel(emb_ref, w_ref, b_ref, out_ref):
    x = emb_ref[...]
    n = jnp.sqrt(jnp.sum(x * x, axis=1, keepdims=True))
    x = x / jnp.maximum(n, _EPS)
    x = jnp.tanh(x)
    h = lax.dot_general(x, w_ref[...], (((1,), (1,)), ((), ())),
                        preferred_element_type=jnp.float32)
    h = h + b_ref[...]
    hn = jnp.sqrt(jnp.sum(h * h, axis=1, keepdims=True))
    h = h / jnp.maximum(hn, _EPS)
    out_ref[...] = jnp.maximum(h, 0.0)


def _tc_dense(emb, w, b2, block_n):
    n_rows, d = emb.shape
    out_dim = w.shape[0]
    grid = (pl.cdiv(n_rows, block_n),)
    return pl.pallas_call(
        _tc_dense_kernel,
        grid=grid,
        in_specs=[
            pl.BlockSpec((block_n, d), lambda i: (i, 0)),
            pl.BlockSpec((out_dim, d), lambda i: (0, 0)),
            pl.BlockSpec((1, out_dim), lambda i: (0, 0)),
        ],
        out_specs=pl.BlockSpec((block_n, out_dim), lambda i: (i, 0)),
        out_shape=jax.ShapeDtypeStruct((n_rows, out_dim), jnp.float32),
    )(emb, w, b2)


@jax.jit
def kernel(indices, table, W, b):
    n = indices.shape[0]
    out_dim = W.shape[0]
    b2 = b.reshape(1, out_dim)
    t4 = _tc_dense(table, W, b2, block_n=2048)
    idx2 = indices.astype(jnp.int32).reshape(n // _STREAM, _STREAM)
    return _sc_gather(idx2, t4)


# speedup vs baseline: 7.2991x; 1.0002x over previous
import functools

import jax
import jax.numpy as jnp
from jax import lax
from jax.experimental import pallas as pl
from jax.experimental.pallas import tpu as pltpu
from jax.experimental.pallas import tpu_sc as plsc

_EPS = 1e-12

_NC = 2
_NS = 16
_NW = _NC * _NS
_STREAM = 128


def _sc_gather_kernel(n_streams, idx_hbm, table_hbm, out_hbm, idx_v,
                      r0, r1, r2, r3, g0, g1, g2, g3):
    wid = lax.axis_index("s") * _NC + lax.axis_index("c")
    row0 = wid * n_streams
    pltpu.sync_copy(idx_hbm.at[pl.ds(row0, n_streams)], idx_v)

    def fire(j, buf, sem):
        pltpu.async_copy(table_hbm.at[idx_v.at[j]], buf, sem)

    def fire_if(j, buf, sem):
        @pl.when(j < n_streams)
        def _():
            fire(j, buf, sem)

    def drain_write(j, buf, sem):
        pltpu.make_async_copy(table_hbm.at[idx_v.at[0]], buf, sem).wait()
        pltpu.sync_copy(buf, out_hbm.at[pl.ds((row0 + j) * _STREAM,
                                              _STREAM)])

    fire(0, r0, g0)
    fire(1, r1, g1)

    def quad(q, carry):
        j = 4 * q
        fire(j + 2, r2, g2)
        drain_write(j, r0, g0)
        fire(j + 3, r3, g3)
        drain_write(j + 1, r1, g1)
        fire_if(j + 4, r0, g0)
        drain_write(j + 2, r2, g2)
        fire_if(j + 5, r1, g1)
        drain_write(j + 3, r3, g3)
        return carry

    lax.fori_loop(0, n_streams // 4, quad, 0, unroll=False)


def _sc_gather(idx2, table):
    n_rows = idx2.shape[0] * idx2.shape[1]
    d = table.shape[1]
    n_streams = n_rows // (_NW * _STREAM)
    mesh = plsc.VectorSubcoreMesh(core_axis_name="c", subcore_axis_name="s")
    kern = pl.kernel(
        functools.partial(_sc_gather_kernel, n_streams),
        out_type=jax.ShapeDtypeStruct((n_rows, d), jnp.float32),
        mesh=mesh,
        scratch_types=[
            pltpu.VMEM((n_streams, _STREAM), jnp.int32),
            pltpu.VMEM((_STREAM, d), jnp.float32),
            pltpu.VMEM((_STREAM, d), jnp.float32),
            pltpu.VMEM((_STREAM, d), jnp.float32),
            pltpu.VMEM((_STREAM, d), jnp.float32),
            pltpu.SemaphoreType.DMA,
            pltpu.SemaphoreType.DMA,
            pltpu.SemaphoreType.DMA,
            pltpu.SemaphoreType.DMA,
        ],
        compiler_params=pltpu.CompilerParams(use_tc_tiling_on_sc=True),
    )
    return kern(idx2, table)


def _tc_dense_kernel(emb_ref, w_ref, b_ref, out_ref):
    x = emb_ref[...]
    n = jnp.sqrt(jnp.sum(x * x, axis=1, keepdims=True))
    x = x / jnp.maximum(n, _EPS)
    x = jnp.tanh(x)
    h = lax.dot_general(x, w_ref[...], (((1,), (1,)), ((), ())),
                        preferred_element_type=jnp.float32)
    h = h + b_ref[...]
    hn = jnp.sqrt(jnp.sum(h * h, axis=1, keepdims=True))
    h = h / jnp.maximum(hn, _EPS)
    out_ref[...] = jnp.maximum(h, 0.0)


def _tc_dense(emb, w, b2, block_n):
    n_rows, d = emb.shape
    out_dim = w.shape[0]
    grid = (pl.cdiv(n_rows, block_n),)
    return pl.pallas_call(
        _tc_dense_kernel,
        grid=grid,
        in_specs=[
            pl.BlockSpec((block_n, d), lambda i: (i, 0)),
            pl.BlockSpec((out_dim, d), lambda i: (0, 0)),
            pl.BlockSpec((1, out_dim), lambda i: (0, 0)),
        ],
        out_specs=pl.BlockSpec((block_n, out_dim), lambda i: (i, 0)),
        out_shape=jax.ShapeDtypeStruct((n_rows, out_dim), jnp.float32),
    )(emb, w, b2)


@jax.jit
def kernel(indices, table, W, b):
    n = indices.shape[0]
    out_dim = W.shape[0]
    b2 = b.reshape(1, out_dim)
    t4 = _tc_dense(table, W, b2, block_n=2048)
    idx2 = indices.astype(jnp.int32).reshape(n // _STREAM, _STREAM)
    return _sc_gather(idx2, t4)


# speedup vs baseline: 9.1022x; 1.2470x over previous
import functools

import jax
import jax.numpy as jnp
from jax import lax
from jax.experimental import pallas as pl
from jax.experimental.pallas import tpu as pltpu
from jax.experimental.pallas import tpu_sc as plsc

_EPS = 1e-12

_NC = 2
_NS = 16
_NW = _NC * _NS
_STREAM = 128


def _sc_gather_kernel(n_streams, idx_hbm, table_hbm, out_hbm, idx_v,
                      r0, r1, r2, r3, g0, g1, g2, g3):
    wid = lax.axis_index("s") * _NC + lax.axis_index("c")
    row0 = wid * n_streams
    pltpu.sync_copy(idx_hbm.at[pl.ds(row0, n_streams)], idx_v)

    def fire(j, buf, sem):
        pltpu.async_copy(table_hbm.at[idx_v.at[j]], buf, sem)

    def fire_if(j, buf, sem):
        @pl.when(j < n_streams)
        def _():
            fire(j, buf, sem)

    def drain_write(j, buf, sem):
        pltpu.make_async_copy(table_hbm.at[idx_v.at[0]], buf, sem).wait()
        pltpu.sync_copy(buf, out_hbm.at[pl.ds((row0 + j) * _STREAM,
                                              _STREAM)])

    fire(0, r0, g0)
    fire(1, r1, g1)

    def quad(q, carry):
        j = 4 * q
        fire(j + 2, r2, g2)
        drain_write(j, r0, g0)
        fire(j + 3, r3, g3)
        drain_write(j + 1, r1, g1)
        fire_if(j + 4, r0, g0)
        drain_write(j + 2, r2, g2)
        fire_if(j + 5, r1, g1)
        drain_write(j + 3, r3, g3)
        return carry

    lax.fori_loop(0, n_streams // 4, quad, 0, unroll=False)


def _sc_gather(idx2, table):
    n_rows = idx2.shape[0] * idx2.shape[1]
    d = table.shape[1]
    n_streams = n_rows // (_NW * _STREAM)
    mesh = plsc.VectorSubcoreMesh(core_axis_name="c", subcore_axis_name="s")
    kern = pl.kernel(
        functools.partial(_sc_gather_kernel, n_streams),
        out_type=jax.ShapeDtypeStruct((n_rows, d), jnp.float32),
        mesh=mesh,
        scratch_types=[
            pltpu.VMEM((n_streams, _STREAM), jnp.int32),
            pltpu.VMEM((_STREAM, d), jnp.float32),
            pltpu.VMEM((_STREAM, d), jnp.float32),
            pltpu.VMEM((_STREAM, d), jnp.float32),
            pltpu.VMEM((_STREAM, d), jnp.float32),
            pltpu.SemaphoreType.DMA,
            pltpu.SemaphoreType.DMA,
            pltpu.SemaphoreType.DMA,
            pltpu.SemaphoreType.DMA,
        ],
        compiler_params=pltpu.CompilerParams(use_tc_tiling_on_sc=True),
    )
    return kern(idx2, table)


def _tc_dense_kernel(embt_ref, wt_ref, b_ref, out_ref):
    x = embt_ref[...]
    n = jnp.sqrt(jnp.sum(x * x, axis=0, keepdims=True))
    x = x / jnp.maximum(n, _EPS)
    x = jnp.tanh(x)
    h = lax.dot_general(x, wt_ref[...], (((0,), (0,)), ((), ())),
                        preferred_element_type=jnp.float32)
    h = h + b_ref[...]
    hn = jnp.sqrt(jnp.sum(h * h, axis=1, keepdims=True))
    h = h / jnp.maximum(hn, _EPS)
    out_ref[...] = jnp.maximum(h, 0.0)


def _tc_dense(embt, wt, b2, block_n):
    d, n_rows = embt.shape
    out_dim = wt.shape[1]
    grid = (pl.cdiv(n_rows, block_n),)
    return pl.pallas_call(
        _tc_dense_kernel,
        grid=grid,
        in_specs=[
            pl.BlockSpec((d, block_n), lambda i: (0, i)),
            pl.BlockSpec((d, out_dim), lambda i: (0, 0)),
            pl.BlockSpec((1, out_dim), lambda i: (0, 0)),
        ],
        out_specs=pl.BlockSpec((block_n, out_dim), lambda i: (i, 0)),
        out_shape=jax.ShapeDtypeStruct((n_rows, out_dim), jnp.float32),
        compiler_params=pltpu.CompilerParams(
            dimension_semantics=("arbitrary",)),
    )(embt, wt, b2)


@jax.jit
def kernel(indices, table, W, b):
    n = indices.shape[0]
    out_dim = W.shape[0]
    b2 = b.reshape(1, out_dim)
    t4 = _tc_dense(table.T, W.T, b2, block_n=2048)
    idx2 = indices.astype(jnp.int32).reshape(n // _STREAM, _STREAM)
    return _sc_gather(idx2, t4)
